# Initial kernel scaffold; baseline (speedup 1.0000x reference)
import jax
import jax.numpy as jnp
from jax.experimental import pallas as pl


def kernel(E_re, E_im, E2_re, E2_im, R_re, R_im, Rs_re, Rs_im, Ro_re, Ro_im, Ts_re, Ts_im, To_re, To_im, s, r, o, t):
    raise NotImplementedError("write your pallas kernel here")


# speedup vs baseline: 1.0485x; 1.0485x over previous
import functools

import jax
import jax.numpy as jnp
from jax import lax
from jax.experimental import pallas as pl
from jax.experimental.pallas import tpu as pltpu
from jax.experimental.pallas import tpu_sc as plsc

N_ENT = 100000
N_REL = 200
N_TIME = 365
D = 200
B = 16384

NC = 2
NS = 16
L = 16
NW = NC * NS
PER_W = B // NW
CB = 32
N_CHUNK = PER_W // CB

FULL_CHUNKS = D // L
TAIL_OFF = D - L
RCOLS = 6 * D
TCOLS = 4 * D


def _score_kernel(e_re, e_im, e2_re, e2_im, rcat, tcat, s_idx, r_idx,
                  o_idx, t_idx, out, s_iv, r_iv, o_iv, t_iv,
                  b_sre, b_sim, b_s2re, b_s2im, b_ore, b_oim, b_o2re,
                  b_o2im, b_r, b_t, out_v, sem):
    wid = lax.axis_index("s") * NC + lax.axis_index("c")
    base = pl.multiple_of(wid * PER_W, 8)

    lane = lax.broadcasted_iota(jnp.int32, (L,), 0)
    tail_mask = lane >= (L - (D - FULL_CHUNKS * L))
    perms = [((lane + sh) % L)[:, None] for sh in (8, 4, 2, 1)]
    _dn = lax.GatherDimensionNumbers(
        offset_dims=(), collapsed_slice_dims=(0,), start_index_map=(0,))

    def _lane_sum(v):
        for p in perms:
            v = v + lax.gather(v, p, _dn, slice_sizes=(1,),
                               mode=lax.GatherScatterMode.PROMISE_IN_BOUNDS)
        return v

    def chunk_body(ch, _):
        off = pl.multiple_of(base + ch * CB, 8)
        pltpu.sync_copy(s_idx.at[pl.ds(off, CB)], s_iv)
        pltpu.sync_copy(r_idx.at[pl.ds(off, CB)], r_iv)
        pltpu.sync_copy(o_idx.at[pl.ds(off, CB)], o_iv)
        pltpu.sync_copy(t_idx.at[pl.ds(off, CB)], t_iv)
        cps = [
            pltpu.async_copy(e_re.at[s_iv], b_sre, sem),
            pltpu.async_copy(e_im.at[s_iv], b_sim, sem),
            pltpu.async_copy(e2_re.at[s_iv], b_s2re, sem),
            pltpu.async_copy(e2_im.at[s_iv], b_s2im, sem),
            pltpu.async_copy(e_re.at[o_iv], b_ore, sem),
            pltpu.async_copy(e_im.at[o_iv], b_oim, sem),
            pltpu.async_copy(e2_re.at[o_iv], b_o2re, sem),
            pltpu.async_copy(e2_im.at[o_iv], b_o2im, sem),
            pltpu.async_copy(rcat.at[r_iv], b_r, sem),
            pltpu.async_copy(tcat.at[t_iv], b_t, sem),
        ]
        for cp in cps:
            cp.wait()

        def elem_body(e, vec):
            acc1 = jnp.zeros((L,), jnp.float32)
            acc5 = jnp.zeros((L,), jnp.float32)
            for c in range(FULL_CHUNKS + 1):
                d0 = TAIL_OFF if c == FULL_CHUNKS else c * L
                s_re = b_sre[e, pl.ds(d0, L)]
                s_im = b_sim[e, pl.ds(d0, L)]
                o_re = b_ore[e, pl.ds(d0, L)]
                o_im = b_oim[e, pl.ds(d0, L)]
                s2_re = b_s2re[e, pl.ds(d0, L)]
                s2_im = b_s2im[e, pl.ds(d0, L)]
                o2_re = b_o2re[e, pl.ds(d0, L)]
                o2_im = b_o2im[e, pl.ds(d0, L)]
                r_re = b_r[e, pl.ds(d0, L)]
                r_im = b_r[e, pl.ds(D + d0, L)]
                rs_re = b_r[e, pl.ds(2 * D + d0, L)]
                rs_im = b_r[e, pl.ds(3 * D + d0, L)]
                ro_re = b_r[e, pl.ds(4 * D + d0, L)]
                ro_im = b_r[e, pl.ds(5 * D + d0, L)]
                ts_re = b_t[e, pl.ds(d0, L)]
                ts_im = b_t[e, pl.ds(D + d0, L)]
                to_re = b_t[e, pl.ds(2 * D + d0, L)]
                to_im = b_t[e, pl.ds(3 * D + d0, L)]
                sro = ((s_im * r_re + s_re * r_im) * o_im
                       + (s_re * r_re - s_im * r_im) * o_re)
                srt = ((s_im * rs_re + s_re * rs_im) * ts_im
                       + (s_re * rs_re - s_im * rs_im) * ts_re)
                ort = ((o_im * ro_re + o_re * ro_im) * to_im
                       + (o_re * ro_re - o_im * ro_im) * to_re)
                sot = ((s2_im * ts_re + s2_re * ts_im) * o2_im
                       + (s2_re * ts_re - s2_im * ts_im) * o2_re)
                w5 = srt + ort + sot
                if c == FULL_CHUNKS:
                    sro = jnp.where(tail_mask, sro, 0.0)
                    w5 = jnp.where(tail_mask, w5, 0.0)
                acc1 = acc1 + sro
                acc5 = acc5 + w5
            tot = _lane_sum(acc1 + 5.0 * acc5)
            return jnp.where(lane == (e % L), tot, vec)

        def group_body(g, _):
            e0 = g * L
            vec = lax.fori_loop(e0, e0 + L, elem_body,
                                jnp.zeros((L,), jnp.float32), unroll=False)
            off = pl.multiple_of(ch * CB + e0, L)
            out_v[pl.ds(off, L)] = vec
            return ()

        lax.fori_loop(0, CB // L, group_body, (), unroll=False)
        return ()

    lax.fori_loop(0, N_CHUNK, chunk_body, (), unroll=False)
    pltpu.sync_copy(out_v, out.at[pl.ds(base, PER_W)])


@jax.jit
def _timeplex_sc(e_re, e_im, e2_re, e2_im, rcat, tcat, s, r, o, t):
    mesh = plsc.VectorSubcoreMesh(core_axis_name="c", subcore_axis_name="s")
    kfn = functools.partial(
        pl.kernel,
        mesh=mesh,
        out_type=jax.ShapeDtypeStruct((B,), jnp.float32),
        scratch_types=[
            pltpu.VMEM((CB,), jnp.int32),
            pltpu.VMEM((CB,), jnp.int32),
            pltpu.VMEM((CB,), jnp.int32),
            pltpu.VMEM((CB,), jnp.int32),
            pltpu.VMEM((CB, D), jnp.float32),
            pltpu.VMEM((CB, D), jnp.float32),
            pltpu.VMEM((CB, D), jnp.float32),
            pltpu.VMEM((CB, D), jnp.float32),
            pltpu.VMEM((CB, D), jnp.float32),
            pltpu.VMEM((CB, D), jnp.float32),
            pltpu.VMEM((CB, D), jnp.float32),
            pltpu.VMEM((CB, D), jnp.float32),
            pltpu.VMEM((CB, RCOLS), jnp.float32),
            pltpu.VMEM((CB, TCOLS), jnp.float32),
            pltpu.VMEM((PER_W,), jnp.float32),
            pltpu.SemaphoreType.DMA,
        ],
        compiler_params=pltpu.CompilerParams(use_tc_tiling_on_sc=False),
    )(_score_kernel)
    return kfn(e_re, e_im, e2_re, e2_im, rcat, tcat, s, r, o, t)


def kernel(E_re, E_im, E2_re, E2_im, R_re, R_im, Rs_re, Rs_im, Ro_re,
           Ro_im, Ts_re, Ts_im, To_re, To_im, s, r, o, t):
    rcat = jnp.concatenate([R_re, R_im, Rs_re, Rs_im, Ro_re, Ro_im], axis=1)
    tcat = jnp.concatenate([Ts_re, Ts_im, To_re, To_im], axis=1)
    return _timeplex_sc(E_re, E_im, E2_re, E2_im, rcat, tcat, s, r, o, t)
